# baseline (speedup 1.0000x reference)
import functools

import jax
import jax.numpy as jnp
from jax import lax
from jax.experimental import pallas as pl
from jax.experimental.pallas import tpu as pltpu
from jax.experimental.pallas import tpu_sc as plsc

N = 10000
E = 320000
D = 128
DH = D // 2

NC = 2
NS = 16
NT = NC * NS
CHUNK = 128
CPT = 80
E_PAD = NT * CPT * CHUNK
NP = 10240

ROWS_PER_TILE = NP // NS

IB = 8
KB = CPT // IB

DCPT = E_PAD // (NT * CHUNK)


def _sc_agg_body(yy_hbm, src_hbm, dst_hbm, psum_hbm,
                 srcb0, srcb1, dstb0, dstb1, buf0, buf1, acc,
                 sem_g0, sem_g1, sem_s0, sem_s1,
                 sem_is0, sem_is1, sem_id0, sem_id1):
    bufs = (buf0, buf1)
    srcb = (srcb0, srcb1)
    dstb = (dstb0, dstb1)
    sem_g = (sem_g0, sem_g1)
    sem_s = (sem_s0, sem_s1)
    sem_is = (sem_is0, sem_is1)
    sem_id = (sem_id0, sem_id1)
    c = lax.axis_index("c")
    s = lax.axis_index("s")
    wid = c * NS + s
    y_hbm = yy_hbm.at[c]
    zero16 = jnp.zeros((16,), jnp.float32)

    def _zrow(r, _):
        for l in range(D // 16):
            buf0[r, pl.ds(l * 16, 16)] = zero16
        return 0
    lax.fori_loop(0, CHUNK, _zrow, 0)

    pltpu.sync_copy(src_hbm.at[pl.ds(wid * CPT, IB)], srcb0)
    pltpu.sync_copy(dst_hbm.at[pl.ds(wid * CPT, IB)], dstb0)

    rows = pl.ds(s * ROWS_PER_TILE, ROWS_PER_TILE)

    def _zacc(j, _):
        pltpu.sync_copy(
            buf0, acc.at[pl.ds(s * ROWS_PER_TILE + j * CHUNK, CHUNK)])
        return 0

    lax.fori_loop(0, ROWS_PER_TILE // CHUNK, _zacc, 0)

    def _gather_start(q, r, b):
        pltpu.async_copy(y_hbm.at[srcb[q].at[r]], bufs[b], sem_g[b])

    def _gather_wait(q, r, b):
        pltpu.make_async_copy(y_hbm.at[srcb[q].at[r]], bufs[b], sem_g[b]).wait()

    def _scatter_wait(b):
        pltpu.make_async_copy(bufs[b], acc.at[dstb0.at[0]], sem_s[b]).wait()

    plsc.subcore_barrier()
    _gather_start(0, 0, 0)

    def _block(B, pb):
        qb = 1 - pb
        @pl.when(B >= 1)
        def _():
            pltpu.make_async_copy(
                dst_hbm.at[pl.ds(wid * CPT, IB)], dstb[pb], sem_id[pb]).wait()

        for r in range(IB):
            b = r % 2
            _gather_wait(pb, r, b)

            if r + 1 < IB:
                @pl.when(B * IB + r >= 1)
                def _():
                    _scatter_wait(1 - b)
                _gather_start(pb, r + 1, 1 - b)
            else:
                @pl.when(B + 1 < KB)
                def _():
                    _scatter_wait(1 - b)
                    pltpu.make_async_copy(
                        src_hbm.at[pl.ds(wid * CPT, IB)], srcb[qb],
                        sem_is[qb]).wait()
                    _gather_start(qb, 0, 1 - b)

            pltpu.async_copy(bufs[b], acc.at[dstb[pb].at[r]], sem_s[b],
                             add=True)

            if r == 0:
                @pl.when(B + 1 < KB)
                def _():
                    pltpu.async_copy(
                        src_hbm.at[pl.ds(wid * CPT + (B + 1) * IB, IB)],
                        srcb[qb], sem_is[qb])
                    pltpu.async_copy(
                        dst_hbm.at[pl.ds(wid * CPT + (B + 1) * IB, IB)],
                        dstb[qb], sem_id[qb])

    def _super(t, _):
        _block(2 * t, 0)
        _block(2 * t + 1, 1)
        return 0

    lax.fori_loop(0, KB // 2, _super, 0)

    _scatter_wait(0)
    _scatter_wait(1)
    plsc.subcore_barrier()

    pltpu.sync_copy(acc.at[rows], psum_hbm.at[c].at[rows])


_sc_agg = pl.kernel(
    _sc_agg_body,
    out_type=[jax.ShapeDtypeStruct((NC, NP, D), jnp.float32)],
    mesh=plsc.VectorSubcoreMesh(core_axis_name="c", subcore_axis_name="s"),
    scratch_types=[
        pltpu.VMEM((IB, CHUNK), jnp.int32),
        pltpu.VMEM((IB, CHUNK), jnp.int32),
        pltpu.VMEM((IB, CHUNK), jnp.int32),
        pltpu.VMEM((IB, CHUNK), jnp.int32),
        pltpu.VMEM((CHUNK, D), jnp.float32),
        pltpu.VMEM((CHUNK, D), jnp.float32),
        pltpu.VMEM_SHARED((NP, D), jnp.float32),
    ] + [pltpu.SemaphoreType.DMA] * 8,
    compiler_params=pltpu.CompilerParams(needs_layout_passes=False,
                                         use_tc_tiling_on_sc=False),
    name="sc_segment_sum",
)


def _sc_deg_body(dst_hbm, deg_hbm, dst_v, deg_v, degblk_v, degsum_v, deg_sh):
    c = lax.axis_index("c")
    s = lax.axis_index("s")
    wid = c * NS + s
    zero16 = jnp.zeros((16,), jnp.float32)
    ones16 = jnp.ones((16,), jnp.float32)

    pltpu.sync_copy(dst_hbm.at[pl.ds(wid * DCPT, DCPT)], dst_v)

    def _zdeg(i, _):
        deg_v[pl.ds(i * 16, 16)] = zero16
        return 0
    lax.fori_loop(0, NP // 16, _zdeg, 0)

    def _chunk(j, _):
        for l in range(CHUNK // 16):
            idx = dst_v[j, pl.ds(l * 16, 16)]
            plsc.addupdate_scatter(deg_v, [idx], ones16)
        return 0
    lax.fori_loop(0, DCPT, _chunk, 0)

    pltpu.sync_copy(deg_v, deg_sh.at[s])
    plsc.subcore_barrier()
    pltpu.sync_copy(deg_sh.at[:, pl.ds(s * ROWS_PER_TILE, ROWS_PER_TILE)],
                    degblk_v)

    def _red(o, _):
        tot = degblk_v[0, pl.ds(o * 16, 16)]
        for r in range(1, NS):
            tot = tot + degblk_v[r, pl.ds(o * 16, 16)]
        degsum_v[pl.ds(o * 16, 16)] = tot
        return 0
    lax.fori_loop(0, ROWS_PER_TILE // 16, _red, 0)

    pltpu.sync_copy(
        degsum_v, deg_hbm.at[pl.ds(c * NP + s * ROWS_PER_TILE, ROWS_PER_TILE)])


_sc_deg = pl.kernel(
    _sc_deg_body,
    out_type=[jax.ShapeDtypeStruct((NC * NP,), jnp.float32)],
    mesh=plsc.VectorSubcoreMesh(core_axis_name="c", subcore_axis_name="s"),
    scratch_types=[
        pltpu.VMEM((DCPT, CHUNK), jnp.int32),
        pltpu.VMEM((NP,), jnp.float32),
        pltpu.VMEM((NS, ROWS_PER_TILE), jnp.float32),
        pltpu.VMEM((ROWS_PER_TILE,), jnp.float32),
        pltpu.VMEM_SHARED((NS, NP), jnp.float32),
    ],
    compiler_params=pltpu.CompilerParams(needs_layout_passes=False),
    name="sc_degree",
)

BR = 1024


def _mm_bias_body(copy_table, x_ref, w_ref, b_ref, *o_refs):
    x = x_ref[0]
    o_refs[0][...] = (
        lax.dot_general(x, w_ref[...], (((1,), (1,)), ((), ())),
                        preferred_element_type=jnp.float32)
        + b_ref[...]
    )
    if copy_table:
        o_refs[1][0] = x
        o_refs[1][1] = x


def _mm_bias(xh, w, b, copy_table=False):
    out_specs = [pl.BlockSpec((BR, D), lambda i: (i, 0))]
    out_shape = [jax.ShapeDtypeStruct((NP, D), jnp.float32)]
    if copy_table:
        out_specs.append(pl.BlockSpec((NC, BR, D), lambda i: (0, i, 0)))
        out_shape.append(jax.ShapeDtypeStruct((NC, NP, D), jnp.float32))
    return pl.pallas_call(
        functools.partial(_mm_bias_body, copy_table),
        grid=(NP // BR,),
        in_specs=[
            pl.BlockSpec((1, BR, D), lambda i: (0, i, 0)),
            pl.BlockSpec((D, D), lambda i: (0, 0)),
            pl.BlockSpec((1, D), lambda i: (0, 0)),
        ],
        out_specs=out_specs,
        out_shape=out_shape,
    )(xh, w, b.reshape(1, D))


def _combine_body(final, psum_ref, deg_ref, z_ref, w_ref, o_ref):
    p = psum_ref[0] + psum_ref[1]
    recip = 1.0 / jnp.maximum(deg_ref[...], 1.0)
    agg = p * recip
    a = lax.dot_general(agg, w_ref[...], (((1,), (1,)), ((), ())),
                        preferred_element_type=jnp.float32) + z_ref[...]
    if final:
        m = jnp.max(a, axis=1, keepdims=True)
        lse = jnp.log(jnp.sum(jnp.exp(a - m), axis=1, keepdims=True)) + m
        o_ref[...] = a - lse
    else:
        h = jnp.maximum(a, 0.0)
        o_ref[0] = h
        o_ref[1] = h


def _combine(final, psum, deg, z, w):
    if final:
        out_specs = pl.BlockSpec((BR, D), lambda i: (i, 0))
        out_shape = jax.ShapeDtypeStruct((NP, D), jnp.float32)
    else:
        out_specs = pl.BlockSpec((NC, BR, D), lambda i: (0, i, 0))
        out_shape = jax.ShapeDtypeStruct((NC, NP, D), jnp.float32)
    return pl.pallas_call(
        functools.partial(_combine_body, final),
        grid=(NP // BR,),
        in_specs=[
            pl.BlockSpec((NC, BR, D), lambda i: (0, i, 0)),
            pl.BlockSpec((BR, 1), lambda i: (i, 0)),
            pl.BlockSpec((BR, D), lambda i: (i, 0)),
            pl.BlockSpec((D, D), lambda i: (0, 0)),
        ],
        out_specs=out_specs,
        out_shape=out_shape,
    )(psum, deg, z, w)


def kernel(x, edge_index, W_l0, b_l0, W_r0, W_l1, b_l1, W_r1):
    src = edge_index[0].astype(jnp.int32)
    dst = edge_index[1].astype(jnp.int32)
    pad = E_PAD - E
    src_p = jnp.concatenate([src, jnp.zeros((pad,), jnp.int32)]).reshape(NT * CPT, CHUNK)
    dst_p = jnp.concatenate([dst, jnp.full((pad,), N, jnp.int32)]).reshape(NT * CPT, CHUNK)

    xp = jnp.pad(x, ((0, NP - N), (0, 0))).reshape(1, NP, D)

    (degp,) = _sc_deg(dst_p)
    degp = degp.reshape(NC, NP)
    deg = (degp[0] + degp[1]).reshape(NP, 1)

    z0, yy0 = _mm_bias(xp, W_r0, b_l0, copy_table=True)
    (psum0,) = _sc_agg(yy0, src_p, dst_p)
    hh = _combine(False, psum0, deg, z0, W_l0)

    (psum1,) = _sc_agg(hh, src_p, dst_p)
    (z1,) = _mm_bias(hh, W_r1, b_l1)
    return _combine(True, psum1, deg, z1, W_l1)[:N]


# speedup vs baseline: 2.9478x; 2.9478x over previous
import functools

import jax
import jax.numpy as jnp
from jax import lax
from jax.experimental import pallas as pl
from jax.experimental.pallas import tpu as pltpu
from jax.experimental.pallas import tpu_sc as plsc

N = 10000
E = 320000
D = 128
DH = D // 2

NC = 2
NS = 16
NT = NC * NS
CHUNK = 128
CPT = 160
E_PAD = NS * CPT * CHUNK
NP = 10240

ROWS_PER_TILE = NP // NS

IB = 8
KB = CPT // IB

DCPT = E_PAD // (NT * CHUNK)


NBUF = 4
LOOK = 2
NSH = 10016
SRT = NSH // NS


def _sc_agg_body(yy_hbm, src_hbm, dst_hbm, psum_hbm, *rest):
    srcb = rest[0:2]
    dstb = rest[2:4]
    bufs = rest[4:4 + NBUF]
    acc = rest[4 + NBUF]
    y_sh = rest[5 + NBUF]
    sem_g = rest[6 + NBUF:6 + 2 * NBUF]
    sem_s = rest[6 + 2 * NBUF:6 + 3 * NBUF]
    sem_is = rest[6 + 3 * NBUF:8 + 3 * NBUF]
    sem_id = rest[8 + 3 * NBUF:10 + 3 * NBUF]
    buf0 = bufs[0]
    c = lax.axis_index("c")
    s = lax.axis_index("s")
    zero16 = jnp.zeros((16,), jnp.float32)

    pltpu.sync_copy(yy_hbm.at[c].at[pl.ds(s * SRT, SRT)],
                    y_sh.at[pl.ds(s * SRT, SRT)])

    def _zrow(r, _):
        for l in range(DH // 16):
            buf0[r, pl.ds(l * 16, 16)] = zero16
        return 0
    lax.fori_loop(0, CHUNK, _zrow, 0)

    pltpu.sync_copy(src_hbm.at[pl.ds(s * CPT, IB)], srcb[0])
    pltpu.sync_copy(dst_hbm.at[pl.ds(s * CPT, IB)], dstb[0])

    rows = pl.ds(s * ROWS_PER_TILE, ROWS_PER_TILE)

    def _zacc(j, _):
        pltpu.sync_copy(
            buf0, acc.at[pl.ds(s * ROWS_PER_TILE + j * CHUNK, CHUNK)])
        return 0

    lax.fori_loop(0, ROWS_PER_TILE // CHUNK, _zacc, 0)

    def _gather_start(q, r, b):
        pltpu.async_copy(y_sh.at[srcb[q].at[r]], bufs[b], sem_g[b])

    def _gather_wait(q, r, b):
        pltpu.make_async_copy(y_sh.at[srcb[q].at[r]], bufs[b], sem_g[b]).wait()

    def _scatter_wait(b):
        pltpu.make_async_copy(bufs[b], acc.at[dstb[0].at[0]], sem_s[b]).wait()

    plsc.subcore_barrier()
    for b in range(LOOK):
        _gather_start(0, b, b)

    def _block(B, pb):
        qb = 1 - pb
        @pl.when(B >= 1)
        def _():
            pltpu.make_async_copy(
                dst_hbm.at[pl.ds(s * CPT, IB)], dstb[pb], sem_id[pb]).wait()

        for r in range(IB):
            b = r % NBUF
            bn = (r + LOOK) % NBUF

            if r < LOOK:
                @pl.when(B >= 1)
                def _():
                    _scatter_wait(bn)
                _gather_start(pb, r + LOOK, bn)
            elif r < IB - LOOK:
                _scatter_wait(bn)
                _gather_start(pb, r + LOOK, bn)
            else:
                @pl.when(B + 1 < KB)
                def _():
                    _scatter_wait(bn)
                    if r == IB - LOOK:
                        pltpu.make_async_copy(
                            src_hbm.at[pl.ds(s * CPT, IB)], srcb[qb],
                            sem_is[qb]).wait()
                    _gather_start(qb, r + LOOK - IB, bn)

            _gather_wait(pb, r, b)

            pltpu.async_copy(bufs[b], acc.at[dstb[pb].at[r]], sem_s[b],
                             add=True)

            if r == 0:
                @pl.when(B + 1 < KB)
                def _():
                    pltpu.async_copy(
                        src_hbm.at[pl.ds(s * CPT + (B + 1) * IB, IB)],
                        srcb[qb], sem_is[qb])
            if r == LOOK:
                @pl.when(B + 1 < KB)
                def _():
                    pltpu.async_copy(
                        dst_hbm.at[pl.ds(s * CPT + (B + 1) * IB, IB)],
                        dstb[qb], sem_id[qb])

    def _super(t, _):
        _block(2 * t, 0)
        _block(2 * t + 1, 1)
        return 0

    lax.fori_loop(0, KB // 2, _super, 0)

    for b in range(NBUF):
        _scatter_wait(b)
    plsc.subcore_barrier()

    pltpu.sync_copy(acc.at[rows], psum_hbm.at[c].at[rows])


_sc_agg = pl.kernel(
    _sc_agg_body,
    out_type=[jax.ShapeDtypeStruct((NC, NP, DH), jnp.float32)],
    mesh=plsc.VectorSubcoreMesh(core_axis_name="c", subcore_axis_name="s"),
    scratch_types=[
        pltpu.VMEM((IB, CHUNK), jnp.int32),
        pltpu.VMEM((IB, CHUNK), jnp.int32),
        pltpu.VMEM((IB, CHUNK), jnp.int32),
        pltpu.VMEM((IB, CHUNK), jnp.int32),
    ] + [pltpu.VMEM((CHUNK, DH), jnp.float32) for _ in range(NBUF)] + [
        pltpu.VMEM_SHARED((NP, DH), jnp.float32),
        pltpu.VMEM_SHARED((NSH, DH), jnp.float32),
    ] + [pltpu.SemaphoreType.DMA] * (2 * NBUF + 4),
    compiler_params=pltpu.CompilerParams(needs_layout_passes=False,
                                         use_tc_tiling_on_sc=False),
    name="sc_segment_sum",
)


def _sc_deg_body(dst_hbm, deg_hbm, dst_v, deg_v, degblk_v, degsum_v, deg_sh):
    c = lax.axis_index("c")
    s = lax.axis_index("s")
    wid = c * NS + s
    zero16 = jnp.zeros((16,), jnp.float32)
    ones16 = jnp.ones((16,), jnp.float32)

    pltpu.sync_copy(dst_hbm.at[pl.ds(wid * DCPT, DCPT)], dst_v)

    def _zdeg(i, _):
        deg_v[pl.ds(i * 16, 16)] = zero16
        return 0
    lax.fori_loop(0, NP // 16, _zdeg, 0)

    def _chunk(j, _):
        for l in range(CHUNK // 16):
            idx = dst_v[j, pl.ds(l * 16, 16)]
            plsc.addupdate_scatter(deg_v, [idx], ones16)
        return 0
    lax.fori_loop(0, DCPT, _chunk, 0)

    pltpu.sync_copy(deg_v, deg_sh.at[s])
    plsc.subcore_barrier()
    pltpu.sync_copy(deg_sh.at[:, pl.ds(s * ROWS_PER_TILE, ROWS_PER_TILE)],
                    degblk_v)

    def _red(o, _):
        tot = degblk_v[0, pl.ds(o * 16, 16)]
        for r in range(1, NS):
            tot = tot + degblk_v[r, pl.ds(o * 16, 16)]
        degsum_v[pl.ds(o * 16, 16)] = tot
        return 0
    lax.fori_loop(0, ROWS_PER_TILE // 16, _red, 0)

    pltpu.sync_copy(
        degsum_v, deg_hbm.at[pl.ds(c * NP + s * ROWS_PER_TILE, ROWS_PER_TILE)])


_sc_deg = pl.kernel(
    _sc_deg_body,
    out_type=[jax.ShapeDtypeStruct((NC * NP,), jnp.float32)],
    mesh=plsc.VectorSubcoreMesh(core_axis_name="c", subcore_axis_name="s"),
    scratch_types=[
        pltpu.VMEM((DCPT, CHUNK), jnp.int32),
        pltpu.VMEM((NP,), jnp.float32),
        pltpu.VMEM((NS, ROWS_PER_TILE), jnp.float32),
        pltpu.VMEM((ROWS_PER_TILE,), jnp.float32),
        pltpu.VMEM_SHARED((NS, NP), jnp.float32),
    ],
    compiler_params=pltpu.CompilerParams(needs_layout_passes=False),
    name="sc_degree",
)

BR = 1024


def _mm_bias_body(copy_table, x_ref, w_ref, b_ref, *o_refs):
    x = jnp.concatenate([x_ref[0], x_ref[1]], axis=1)
    o_refs[0][...] = (
        lax.dot_general(x, w_ref[...], (((1,), (1,)), ((), ())),
                        preferred_element_type=jnp.float32)
        + b_ref[...]
    )
    if copy_table:
        o_refs[1][...] = x_ref[...]


def _mm_bias(xh, w, b, copy_table=False):
    out_specs = [pl.BlockSpec((BR, D), lambda i: (i, 0))]
    out_shape = [jax.ShapeDtypeStruct((NP, D), jnp.float32)]
    if copy_table:
        out_specs.append(pl.BlockSpec((NC, BR, DH), lambda i: (0, i, 0)))
        out_shape.append(jax.ShapeDtypeStruct((NC, NP, DH), jnp.float32))
    return pl.pallas_call(
        functools.partial(_mm_bias_body, copy_table),
        grid=(NP // BR,),
        in_specs=[
            pl.BlockSpec((NC, BR, DH), lambda i: (0, i, 0)),
            pl.BlockSpec((D, D), lambda i: (0, 0)),
            pl.BlockSpec((1, D), lambda i: (0, 0)),
        ],
        out_specs=out_specs,
        out_shape=out_shape,
    )(xh, w, b.reshape(1, D))


def _combine_body(final, psum_ref, deg_ref, z_ref, w_ref, o_ref):
    p = jnp.concatenate([psum_ref[0], psum_ref[1]], axis=1)
    recip = 1.0 / jnp.maximum(deg_ref[...], 1.0)
    agg = p * recip
    a = lax.dot_general(agg, w_ref[...], (((1,), (1,)), ((), ())),
                        preferred_element_type=jnp.float32) + z_ref[...]
    if final:
        m = jnp.max(a, axis=1, keepdims=True)
        lse = jnp.log(jnp.sum(jnp.exp(a - m), axis=1, keepdims=True)) + m
        o_ref[...] = a - lse
    else:
        h = jnp.maximum(a, 0.0)
        o_ref[0] = h[:, :DH]
        o_ref[1] = h[:, DH:]


def _combine(final, psum, deg, z, w):
    if final:
        out_specs = pl.BlockSpec((BR, D), lambda i: (i, 0))
        out_shape = jax.ShapeDtypeStruct((NP, D), jnp.float32)
    else:
        out_specs = pl.BlockSpec((NC, BR, DH), lambda i: (0, i, 0))
        out_shape = jax.ShapeDtypeStruct((NC, NP, DH), jnp.float32)
    return pl.pallas_call(
        functools.partial(_combine_body, final),
        grid=(NP // BR,),
        in_specs=[
            pl.BlockSpec((NC, BR, DH), lambda i: (0, i, 0)),
            pl.BlockSpec((BR, 1), lambda i: (i, 0)),
            pl.BlockSpec((BR, D), lambda i: (i, 0)),
            pl.BlockSpec((D, D), lambda i: (0, 0)),
        ],
        out_specs=out_specs,
        out_shape=out_shape,
    )(psum, deg, z, w)


def kernel(x, edge_index, W_l0, b_l0, W_r0, W_l1, b_l1, W_r1):
    src = edge_index[0].astype(jnp.int32)
    dst = edge_index[1].astype(jnp.int32)
    pad = E_PAD - E
    src_p = jnp.concatenate([src, jnp.zeros((pad,), jnp.int32)]).reshape(NS * CPT, CHUNK)
    dst_p = jnp.concatenate([dst, jnp.full((pad,), N, jnp.int32)]).reshape(NS * CPT, CHUNK)

    xp = jnp.pad(x, ((0, NP - N), (0, 0)))
    xh = jnp.stack([xp[:, :DH], xp[:, DH:]])

    (degp,) = _sc_deg(dst_p)
    degp = degp.reshape(NC, NP)
    deg = (degp[0] + degp[1]).reshape(NP, 1)

    z0, xh2 = _mm_bias(xh, W_r0, b_l0, copy_table=True)
    (psum0,) = _sc_agg(xh2, src_p, dst_p)
    hh = _combine(False, psum0, deg, z0, W_l0)

    (psum1,) = _sc_agg(hh, src_p, dst_p)
    (z1,) = _mm_bias(hh, W_r1, b_l1)
    return _combine(True, psum1, deg, z1, W_l1)[:N]


# speedup vs baseline: 3.0602x; 1.0381x over previous
import functools

import jax
import jax.numpy as jnp
from jax import lax
from jax.experimental import pallas as pl
from jax.experimental.pallas import tpu as pltpu
from jax.experimental.pallas import tpu_sc as plsc

N = 10000
E = 320000
D = 128
DH = D // 2

NC = 2
NS = 16
NT = NC * NS
CHUNK = 128
CPT = 160
E_PAD = NS * CPT * CHUNK
NP = 10240

ROWS_PER_TILE = NP // NS

IB = 8
KB = CPT // IB

DCPT = E_PAD // (NT * CHUNK)


NBUF = 4
LOOK = 2
NSH = 10016
SRT = NSH // NS


def _sc_agg_body(yy_hbm, src_hbm, dst_hbm, deg_order_hbm, psum_hbm, *rest):
    srcb = rest[0:2]
    dstb = rest[2:4]
    bufs = rest[4:4 + NBUF]
    acc = rest[4 + NBUF]
    y_sh = rest[5 + NBUF]
    sem_g = rest[6 + NBUF:6 + 2 * NBUF]
    sem_s = rest[6 + 2 * NBUF:6 + 3 * NBUF]
    sem_is = rest[6 + 3 * NBUF:8 + 3 * NBUF]
    sem_id = rest[8 + 3 * NBUF:10 + 3 * NBUF]
    buf0 = bufs[0]
    c = lax.axis_index("c")
    s = lax.axis_index("s")
    zero16 = jnp.zeros((16,), jnp.float32)

    pltpu.sync_copy(yy_hbm.at[c].at[pl.ds(s * SRT, SRT)],
                    y_sh.at[pl.ds(s * SRT, SRT)])

    def _zrow(r, _):
        for l in range(DH // 16):
            buf0[r, pl.ds(l * 16, 16)] = zero16
        return 0
    lax.fori_loop(0, CHUNK, _zrow, 0)

    pltpu.sync_copy(src_hbm.at[pl.ds(s * CPT, IB)], srcb[0])
    pltpu.sync_copy(dst_hbm.at[pl.ds(s * CPT, IB)], dstb[0])

    rows = pl.ds(s * ROWS_PER_TILE, ROWS_PER_TILE)

    def _zacc(j, _):
        pltpu.sync_copy(
            buf0, acc.at[pl.ds(s * ROWS_PER_TILE + j * CHUNK, CHUNK)])
        return 0

    lax.fori_loop(0, ROWS_PER_TILE // CHUNK, _zacc, 0)

    def _gather_start(q, r, b):
        pltpu.async_copy(y_sh.at[srcb[q].at[r]], bufs[b], sem_g[b])

    def _gather_wait(q, r, b):
        pltpu.make_async_copy(y_sh.at[srcb[q].at[r]], bufs[b], sem_g[b]).wait()

    def _scatter_wait(b):
        pltpu.make_async_copy(bufs[b], acc.at[dstb[0].at[0]], sem_s[b]).wait()

    plsc.subcore_barrier()
    for b in range(LOOK):
        _gather_start(0, b, b)

    def _block(B, pb):
        qb = 1 - pb
        @pl.when(B >= 1)
        def _():
            pltpu.make_async_copy(
                dst_hbm.at[pl.ds(s * CPT, IB)], dstb[pb], sem_id[pb]).wait()

        for r in range(IB):
            b = r % NBUF
            bn = (r + LOOK) % NBUF

            if r < LOOK:
                @pl.when(B >= 1)
                def _():
                    _scatter_wait(bn)
                _gather_start(pb, r + LOOK, bn)
            elif r < IB - LOOK:
                _scatter_wait(bn)
                _gather_start(pb, r + LOOK, bn)
            else:
                @pl.when(B + 1 < KB)
                def _():
                    _scatter_wait(bn)
                    if r == IB - LOOK:
                        pltpu.make_async_copy(
                            src_hbm.at[pl.ds(s * CPT, IB)], srcb[qb],
                            sem_is[qb]).wait()
                    _gather_start(qb, r + LOOK - IB, bn)

            _gather_wait(pb, r, b)

            pltpu.async_copy(bufs[b], acc.at[dstb[pb].at[r]], sem_s[b],
                             add=True)

            if r == 0:
                @pl.when(B + 1 < KB)
                def _():
                    pltpu.async_copy(
                        src_hbm.at[pl.ds(s * CPT + (B + 1) * IB, IB)],
                        srcb[qb], sem_is[qb])
            if r == LOOK:
                @pl.when(B + 1 < KB)
                def _():
                    pltpu.async_copy(
                        dst_hbm.at[pl.ds(s * CPT + (B + 1) * IB, IB)],
                        dstb[qb], sem_id[qb])

    def _super(t, _):
        _block(2 * t, 0)
        _block(2 * t + 1, 1)
        return 0

    lax.fori_loop(0, KB // 2, _super, 0)

    for b in range(NBUF):
        _scatter_wait(b)
    plsc.subcore_barrier()

    pltpu.sync_copy(acc.at[rows], psum_hbm.at[c].at[rows])


_sc_agg = pl.kernel(
    _sc_agg_body,
    out_type=[jax.ShapeDtypeStruct((NC, NP, DH), jnp.float32)],
    mesh=plsc.VectorSubcoreMesh(core_axis_name="c", subcore_axis_name="s"),
    scratch_types=[
        pltpu.VMEM((IB, CHUNK), jnp.int32),
        pltpu.VMEM((IB, CHUNK), jnp.int32),
        pltpu.VMEM((IB, CHUNK), jnp.int32),
        pltpu.VMEM((IB, CHUNK), jnp.int32),
    ] + [pltpu.VMEM((CHUNK, DH), jnp.float32) for _ in range(NBUF)] + [
        pltpu.VMEM_SHARED((NP, DH), jnp.float32),
        pltpu.VMEM_SHARED((NSH, DH), jnp.float32),
    ] + [pltpu.SemaphoreType.DMA] * (2 * NBUF + 4),
    compiler_params=pltpu.CompilerParams(needs_layout_passes=False,
                                         use_tc_tiling_on_sc=False),
    name="sc_segment_sum",
)


def _sc_deg_body(dst_hbm, deg_hbm, dst_v, deg_v, degblk_v, degsum_v, deg_sh):
    c = lax.axis_index("c")
    s = lax.axis_index("s")
    wid = c * NS + s
    zero16 = jnp.zeros((16,), jnp.float32)
    ones16 = jnp.ones((16,), jnp.float32)

    pltpu.sync_copy(dst_hbm.at[pl.ds(wid * DCPT, DCPT)], dst_v)

    def _zdeg(i, _):
        deg_v[pl.ds(i * 16, 16)] = zero16
        return 0
    lax.fori_loop(0, NP // 16, _zdeg, 0)

    def _chunk(j, _):
        for l in range(CHUNK // 16):
            idx = dst_v[j, pl.ds(l * 16, 16)]
            plsc.addupdate_scatter(deg_v, [idx], ones16)
        return 0
    lax.fori_loop(0, DCPT, _chunk, 0)

    pltpu.sync_copy(deg_v, deg_sh.at[s])
    plsc.subcore_barrier()
    pltpu.sync_copy(deg_sh.at[:, pl.ds(s * ROWS_PER_TILE, ROWS_PER_TILE)],
                    degblk_v)

    def _red(o, _):
        tot = degblk_v[0, pl.ds(o * 16, 16)]
        for r in range(1, NS):
            tot = tot + degblk_v[r, pl.ds(o * 16, 16)]
        degsum_v[pl.ds(o * 16, 16)] = tot
        return 0
    lax.fori_loop(0, ROWS_PER_TILE // 16, _red, 0)

    pltpu.sync_copy(
        degsum_v, deg_hbm.at[pl.ds(c * NP + s * ROWS_PER_TILE, ROWS_PER_TILE)])


_sc_deg = pl.kernel(
    _sc_deg_body,
    out_type=[jax.ShapeDtypeStruct((NC * NP,), jnp.float32)],
    mesh=plsc.VectorSubcoreMesh(core_axis_name="c", subcore_axis_name="s"),
    scratch_types=[
        pltpu.VMEM((DCPT, CHUNK), jnp.int32),
        pltpu.VMEM((NP,), jnp.float32),
        pltpu.VMEM((NS, ROWS_PER_TILE), jnp.float32),
        pltpu.VMEM((ROWS_PER_TILE,), jnp.float32),
        pltpu.VMEM_SHARED((NS, NP), jnp.float32),
    ],
    compiler_params=pltpu.CompilerParams(needs_layout_passes=False),
    name="sc_degree",
)

BR = 1024


def _mm_bias_body(copy_table, x_ref, w_ref, b_ref, *o_refs):
    x = jnp.concatenate([x_ref[0], x_ref[1]], axis=1)
    o_refs[0][...] = (
        lax.dot_general(x, w_ref[...], (((1,), (1,)), ((), ())),
                        preferred_element_type=jnp.float32)
        + b_ref[...]
    )
    if copy_table:
        o_refs[1][...] = x_ref[...]


def _mm_bias(xh, w, b, copy_table=False):
    out_specs = [pl.BlockSpec((BR, D), lambda i: (i, 0))]
    out_shape = [jax.ShapeDtypeStruct((NP, D), jnp.float32)]
    if copy_table:
        out_specs.append(pl.BlockSpec((NC, BR, DH), lambda i: (0, i, 0)))
        out_shape.append(jax.ShapeDtypeStruct((NC, NP, DH), jnp.float32))
    return pl.pallas_call(
        functools.partial(_mm_bias_body, copy_table),
        grid=(NP // BR,),
        in_specs=[
            pl.BlockSpec((NC, BR, DH), lambda i: (0, i, 0)),
            pl.BlockSpec((D, D), lambda i: (0, 0)),
            pl.BlockSpec((1, D), lambda i: (0, 0)),
        ],
        out_specs=out_specs,
        out_shape=out_shape,
    )(xh, w, b.reshape(1, D))


def _combine_body(final, psum_ref, deg_ref, z_ref, w_ref, *rest):
    p = jnp.concatenate([psum_ref[0], psum_ref[1]], axis=1)
    recip = 1.0 / jnp.maximum(deg_ref[...], 1.0)
    agg = p * recip
    a = lax.dot_general(agg, w_ref[...], (((1,), (1,)), ((), ())),
                        preferred_element_type=jnp.float32) + z_ref[...]
    if final:
        o_ref = rest[0]
        m = jnp.max(a, axis=1, keepdims=True)
        lse = jnp.log(jnp.sum(jnp.exp(a - m), axis=1, keepdims=True)) + m
        o_ref[...] = a - lse
    else:
        w2_ref, b2_ref, o_ref, z2_ref = rest
        h = jnp.maximum(a, 0.0)
        o_ref[0] = h[:, :DH]
        o_ref[1] = h[:, DH:]
        z2_ref[...] = lax.dot_general(
            h, w2_ref[...], (((1,), (1,)), ((), ())),
            preferred_element_type=jnp.float32) + b2_ref[...]


def _combine(final, psum, deg, z, w, w2=None, b2=None):
    in_specs = [
        pl.BlockSpec((NC, BR, DH), lambda i: (0, i, 0)),
        pl.BlockSpec((BR, 1), lambda i: (i, 0)),
        pl.BlockSpec((BR, D), lambda i: (i, 0)),
        pl.BlockSpec((D, D), lambda i: (0, 0)),
    ]
    args = [psum, deg, z, w]
    if final:
        out_specs = pl.BlockSpec((BR, D), lambda i: (i, 0))
        out_shape = jax.ShapeDtypeStruct((NP, D), jnp.float32)
    else:
        in_specs += [pl.BlockSpec((D, D), lambda i: (0, 0)),
                     pl.BlockSpec((1, D), lambda i: (0, 0))]
        args += [w2, b2.reshape(1, D)]
        out_specs = [pl.BlockSpec((NC, BR, DH), lambda i: (0, i, 0)),
                     pl.BlockSpec((BR, D), lambda i: (i, 0))]
        out_shape = [jax.ShapeDtypeStruct((NC, NP, DH), jnp.float32),
                     jax.ShapeDtypeStruct((NP, D), jnp.float32)]
    return pl.pallas_call(
        functools.partial(_combine_body, final),
        grid=(NP // BR,),
        in_specs=in_specs,
        out_specs=out_specs,
        out_shape=out_shape,
    )(*args)


def kernel(x, edge_index, W_l0, b_l0, W_r0, W_l1, b_l1, W_r1):
    src = edge_index[0].astype(jnp.int32)
    dst = edge_index[1].astype(jnp.int32)
    pad = E_PAD - E
    src_p = jnp.concatenate([src, jnp.zeros((pad,), jnp.int32)]).reshape(NS * CPT, CHUNK)
    dst_p = jnp.concatenate([dst, jnp.full((pad,), N, jnp.int32)]).reshape(NS * CPT, CHUNK)

    xp = jnp.pad(x, ((0, NP - N), (0, 0)))
    xh = jnp.stack([xp[:, :DH], xp[:, DH:]])

    (degp,) = _sc_deg(dst_p)
    degp = degp.reshape(NC, NP)
    deg = (degp[0] + degp[1]).reshape(NP, 1)

    z0, xh2 = _mm_bias(xh, W_r0, b_l0, copy_table=True)
    (psum0,) = _sc_agg(xh2, src_p, dst_p, degp)
    hh, z1 = _combine(False, psum0, deg, z0, W_l0, W_r1, b_l1)

    (psum1,) = _sc_agg(hh, src_p, dst_p, degp)
    return _combine(True, psum1, deg, z1, W_l1)[:N]
